# Initial kernel scaffold; baseline (speedup 1.0000x reference)
import jax
import jax.numpy as jnp
from jax.experimental import pallas as pl


def kernel(input_ids, table, W, b):
    raise NotImplementedError("write your pallas kernel here")


# speedup vs baseline: 8.3639x; 8.3639x over previous
import functools

import jax
import jax.numpy as jnp
from jax import lax
from jax.experimental import pallas as pl
from jax.experimental.pallas import tpu as pltpu
from jax.experimental.pallas import tpu_sc as plsc

ROW_BLK = 512
CH = 64


def _mm_kernel(t_ref, w_ref, b_ref, o_ref):
    o_ref[...] = lax.dot_general(
        t_ref[...], w_ref[...], (((1,), (1,)), ((), ())),
        preferred_element_type=jnp.float32) + b_ref[...]


def _transform_table(table, W, b):
    V, D = table.shape
    grid = (pl.cdiv(V, ROW_BLK),)
    return pl.pallas_call(
        _mm_kernel,
        grid=grid,
        in_specs=[
            pl.BlockSpec((ROW_BLK, D), lambda i: (i, 0)),
            pl.BlockSpec((D, D), lambda i: (0, 0)),
            pl.BlockSpec((1, D), lambda i: (0, 0)),
        ],
        out_specs=pl.BlockSpec((ROW_BLK, D), lambda i: (i, 0)),
        out_shape=jax.ShapeDtypeStruct((V, D), jnp.float32),
    )(table, W, b.reshape(1, D))


def _gather_rows(t2, ids_flat):
    info = plsc.get_sparse_core_info()
    NC, NS = info.num_cores, info.num_subcores
    NW = NC * NS
    N = ids_flat.shape[0]
    D = t2.shape[1]
    assert N % (NW * CH) == 0
    b_per_w = N // NW
    n_ch = b_per_w // CH
    mesh = plsc.VectorSubcoreMesh(core_axis_name="c", subcore_axis_name="s")

    @functools.partial(
        pl.kernel,
        mesh=mesh,
        out_type=jax.ShapeDtypeStruct((N, D), jnp.float32),
        scratch_types=[
            pltpu.VMEM((b_per_w,), jnp.int32),
            pltpu.VMEM((CH, D), jnp.float32),
            pltpu.SemaphoreType.DMA,
        ],
    )
    def k(t2_hbm, idx_hbm, out_hbm, idx_v, rows_v, sem):
        wid = lax.axis_index("s") * NC + lax.axis_index("c")
        base = wid * b_per_w
        pltpu.sync_copy(idx_hbm.at[pl.ds(base, b_per_w)], idx_v)

        def body(c, carry):
            off = c * CH
            pltpu.async_copy(
                t2_hbm.at[idx_v.at[pl.ds(off, CH)]], rows_v, sem).wait()
            pltpu.sync_copy(rows_v, out_hbm.at[pl.ds(base + off, CH)])
            return carry

        lax.fori_loop(0, n_ch, body, 0)

    return k(t2, ids_flat)


def kernel(input_ids, table, W, b):
    B, L = input_ids.shape
    t2 = _transform_table(table, W, b)
    ids_flat = input_ids.reshape(B * L).astype(jnp.int32)
    out_flat = _gather_rows(t2, ids_flat)
    return out_flat.reshape(B, L, -1)


# speedup vs baseline: 9.1651x; 1.0958x over previous
import functools

import jax
import jax.numpy as jnp
from jax import lax
from jax.experimental import pallas as pl
from jax.experimental.pallas import tpu as pltpu
from jax.experimental.pallas import tpu_sc as plsc

ROW_BLK = 512
CH = 64


def _mm_kernel(t_ref, w_ref, b_ref, o_ref):
    o_ref[...] = lax.dot_general(
        t_ref[...], w_ref[...], (((1,), (1,)), ((), ())),
        preferred_element_type=jnp.float32) + b_ref[...]


def _transform_table(table, W, b):
    V, D = table.shape
    grid = (pl.cdiv(V, ROW_BLK),)
    return pl.pallas_call(
        _mm_kernel,
        grid=grid,
        in_specs=[
            pl.BlockSpec((ROW_BLK, D), lambda i: (i, 0)),
            pl.BlockSpec((D, D), lambda i: (0, 0)),
            pl.BlockSpec((1, D), lambda i: (0, 0)),
        ],
        out_specs=pl.BlockSpec((ROW_BLK, D), lambda i: (i, 0)),
        out_shape=jax.ShapeDtypeStruct((V, D), jnp.float32),
    )(table, W, b.reshape(1, D))


def _gather_rows(t2, ids_flat):
    info = plsc.get_sparse_core_info()
    NC, NS = info.num_cores, info.num_subcores
    NW = NC * NS
    N = ids_flat.shape[0]
    D = t2.shape[1]
    assert N % (NW * CH) == 0
    b_per_w = N // NW
    n_ch = b_per_w // CH
    mesh = plsc.VectorSubcoreMesh(core_axis_name="c", subcore_axis_name="s")

    n_pairs = n_ch // 2
    assert n_ch % 2 == 0 and n_pairs >= 2

    @functools.partial(
        pl.kernel,
        mesh=mesh,
        out_type=jax.ShapeDtypeStruct((N, D), jnp.float32),
        scratch_types=[
            pltpu.VMEM((b_per_w,), jnp.int32),
            pltpu.VMEM((CH, D), jnp.float32),
            pltpu.VMEM((CH, D), jnp.float32),
            pltpu.SemaphoreType.DMA,
            pltpu.SemaphoreType.DMA,
        ],
    )
    def k(t2_hbm, idx_hbm, out_hbm, idx_v, rows0, rows1, sem0, sem1):
        wid = lax.axis_index("s") * NC + lax.axis_index("c")
        base = wid * b_per_w
        pltpu.sync_copy(idx_hbm.at[pl.ds(base, b_per_w)], idx_v)

        bufs = (rows0, rows1)
        sems = (sem0, sem1)

        def start_gather(c, b):
            pltpu.async_copy(
                t2_hbm.at[idx_v.at[pl.ds(c * CH, CH)]], bufs[b], sems[b])

        def wait_gather(b):
            pltpu.make_async_copy(t2_hbm.at[pl.ds(0, CH)], bufs[b],
                                  sems[b]).wait()

        start_gather(0, 0)
        start_gather(1, 1)

        def body(p, carry):
            for b in range(2):
                c = 2 * p + b
                wait_gather(b)
                pltpu.sync_copy(bufs[b], out_hbm.at[pl.ds(base + c * CH, CH)])
                start_gather(c + 2, b)
            return carry

        lax.fori_loop(0, n_pairs - 1, body, 0)
        for b in range(2):
            c = n_ch - 2 + b
            wait_gather(b)
            pltpu.sync_copy(bufs[b], out_hbm.at[pl.ds(base + c * CH, CH)])

    return k(t2, ids_flat)


def kernel(input_ids, table, W, b):
    B, L = input_ids.shape
    t2 = _transform_table(table, W, b)
    ids_flat = input_ids.reshape(B * L).astype(jnp.int32)
    out_flat = _gather_rows(t2, ids_flat)
    return out_flat.reshape(B, L, -1)
